# baseline (speedup 1.0000x reference)
import jax
import jax.numpy as jnp
from jax import lax
from jax.experimental import pallas as pl
from jax.experimental.pallas import tpu as pltpu
from jax.experimental.pallas import tpu_sc as plsc

_VOCAB = 1000000
_D = 64
_B = 16384
_S = 50
_SCALE = float(_D) ** 0.5

_SP = 56
_DP = 128

_NC = 2
_NSUB = 16
_NW = _NC * _NSUB
_ROWS_PER_W = _B // _NW
_SPAD = 128
_NBUF = 4
_NGRP = _ROWS_PER_W // _NBUF


def _sc_body(idx_hbm, table_hbm, out_hbm, idx_v, *scratch):
    gbufs = scratch[:_NBUF]
    sbufs = scratch[_NBUF:2 * _NBUF]
    gsems = scratch[2 * _NBUF:3 * _NBUF]
    ssems = scratch[3 * _NBUF:]

    wid = lax.axis_index("s") * _NC + lax.axis_index("c")
    row0 = wid * _ROWS_PER_W

    pltpu.sync_copy(idx_hbm.at[pl.ds(row0, _ROWS_PER_W)], idx_v)

    for b in range(_NBUF):
        pltpu.async_copy(
            table_hbm.at[idx_v.at[b, pl.ds(0, _SP)]], gbufs[b], gsems[b]
        )

    @pl.loop(0, _NGRP)
    def _group(g):
        for b in range(_NBUF):
            j = g * _NBUF + b

            pltpu.make_async_copy(
                table_hbm.at[pl.ds(0, _SP)], gbufs[b], gsems[b]
            ).wait()

            @pl.when(g > 0)
            def _():
                pltpu.make_async_copy(
                    sbufs[b], out_hbm.at[0], ssems[b]
                ).wait()

            @plsc.parallel_loop(0, _SP, unroll=8)
            def _row(r):
                for c in range(_D // 16):
                    sbufs[b][r, pl.ds(c * 16, 16)] = (
                        gbufs[b][r, pl.ds(c * 16, 16)] * _SCALE
                    )

            pltpu.async_copy(sbufs[b], out_hbm.at[row0 + j], ssems[b])

            @pl.when(g < _NGRP - 1)
            def _():
                pltpu.async_copy(
                    table_hbm.at[idx_v.at[j + _NBUF, pl.ds(0, _SP)]],
                    gbufs[b],
                    gsems[b],
                )

    for b in range(_NBUF):
        pltpu.make_async_copy(
            sbufs[b], out_hbm.at[0], ssems[b]
        ).wait()


def kernel(inputs, embeddings):
    mesh = plsc.VectorSubcoreMesh(core_axis_name="c", subcore_axis_name="s")
    scratch = (
        [pltpu.VMEM((_ROWS_PER_W, _SPAD), jnp.int32)]
        + [pltpu.VMEM((_SP, _D), jnp.float32) for _ in range(_NBUF)]
        + [pltpu.VMEM((_SP, _DP), jnp.float32) for _ in range(_NBUF)]
        + [pltpu.SemaphoreType.DMA for _ in range(2 * _NBUF)]
    )
    f = pl.kernel(
        _sc_body,
        out_type=jax.ShapeDtypeStruct((_B, _SP, _DP), jnp.float32),
        mesh=mesh,
        scratch_types=scratch,
        compiler_params=pltpu.CompilerParams(use_tc_tiling_on_sc=False),
    )
    idx = jnp.pad(inputs.astype(jnp.int32), ((0, 0), (0, _SPAD - _S)))
    return f(idx, embeddings)[:, :_S, :_D]


# speedup vs baseline: 3.3152x; 3.3152x over previous
import jax
import jax.numpy as jnp
from jax import lax
from jax.experimental import pallas as pl
from jax.experimental.pallas import tpu as pltpu
from jax.experimental.pallas import tpu_sc as plsc

_VOCAB = 1000000
_D = 64
_B = 16384
_S = 50
_SCALE = float(_D) ** 0.5

_SP = 56
_DP = 128

_NC = 2
_NSUB = 16
_NW = _NC * _NSUB
_ROWS_PER_W = _B // _NW
_NBUF = 8
_NGRP = _ROWS_PER_W // _NBUF


def _sc_body(idx_hbm, table_hbm, out_hbm, idx_v, *scratch):
    gbufs = scratch[:_NBUF]
    sbufs = scratch[_NBUF:2 * _NBUF]
    gsems = scratch[2 * _NBUF:3 * _NBUF]
    ssems = scratch[3 * _NBUF:]

    wid = lax.axis_index("s") * _NC + lax.axis_index("c")
    row0 = wid * _ROWS_PER_W

    pltpu.sync_copy(idx_hbm.at[pl.ds(row0, _ROWS_PER_W)], idx_v)

    for b in range(_NBUF):
        pltpu.async_copy(
            table_hbm.at[idx_v.at[b]], gbufs[b], gsems[b]
        )

    @pl.loop(0, _NGRP)
    def _group(g):
        for b in range(_NBUF):
            j = g * _NBUF + b

            pltpu.make_async_copy(
                table_hbm.at[pl.ds(0, _S)], gbufs[b], gsems[b]
            ).wait()

            @pl.when(g > 0)
            def _():
                pltpu.make_async_copy(
                    sbufs[b], out_hbm.at[0], ssems[b]
                ).wait()

            @plsc.parallel_loop(0, _S, unroll=10)
            def _row(r):
                for c in range(_D // 16):
                    sbufs[b][r, pl.ds(c * 16, 16)] = (
                        gbufs[b][r, pl.ds(c * 16, 16)] * _SCALE
                    )

            pltpu.async_copy(sbufs[b], out_hbm.at[row0 + j], ssems[b])

            @pl.when(g < _NGRP - 1)
            def _():
                pltpu.async_copy(
                    table_hbm.at[idx_v.at[j + _NBUF]],
                    gbufs[b],
                    gsems[b],
                )

    for b in range(_NBUF):
        pltpu.make_async_copy(
            sbufs[b], out_hbm.at[0], ssems[b]
        ).wait()


def kernel(inputs, embeddings):
    mesh = plsc.VectorSubcoreMesh(core_axis_name="c", subcore_axis_name="s")
    scratch = (
        [pltpu.VMEM((_ROWS_PER_W, _S), jnp.int32)]
        + [pltpu.VMEM((_S, _D), jnp.float32) for _ in range(_NBUF)]
        + [pltpu.VMEM((_SP, _DP), jnp.float32) for _ in range(_NBUF)]
        + [pltpu.SemaphoreType.DMA for _ in range(2 * _NBUF)]
    )
    f = pl.kernel(
        _sc_body,
        out_type=jax.ShapeDtypeStruct((_B, _SP, _DP), jnp.float32),
        mesh=mesh,
        scratch_types=scratch,
        compiler_params=pltpu.CompilerParams(use_tc_tiling_on_sc=False),
    )
    return f(inputs.astype(jnp.int32), embeddings)[:, :_S, :_D]


# speedup vs baseline: 3.7034x; 1.1171x over previous
import jax
import jax.numpy as jnp
from jax import lax
from jax.experimental import pallas as pl
from jax.experimental.pallas import tpu as pltpu
from jax.experimental.pallas import tpu_sc as plsc

_VOCAB = 1000000
_D = 64
_B = 16384
_S = 50
_SCALE = float(_D) ** 0.5

_SP = 56
_DP = 128

_NC = 2
_NSUB = 16
_NW = _NC * _NSUB
_ROWS_PER_W = _B // _NW
_NBUF = 8
_NGRP = _ROWS_PER_W // _NBUF


def _sc_body(idx_hbm, table_hbm, out_hbm, idx_v, *scratch):
    gbufs = scratch[:_NBUF]
    sbufs = scratch[_NBUF:2 * _NBUF]
    gsems = scratch[2 * _NBUF:3 * _NBUF]
    ssems = scratch[3 * _NBUF:]

    wid = lax.axis_index("s") * _NC + lax.axis_index("c")
    row0 = wid * _ROWS_PER_W

    pltpu.sync_copy(idx_hbm.at[pl.ds(row0, _ROWS_PER_W)], idx_v)

    for b in range(_NBUF):
        pltpu.async_copy(
            table_hbm.at[idx_v.at[b]], gbufs[b], gsems[b]
        )

    @pl.loop(0, _NGRP)
    def _group(g):
        for b in range(_NBUF):
            j = g * _NBUF + b

            pltpu.make_async_copy(
                table_hbm.at[pl.ds(0, _S)], gbufs[b], gsems[b]
            ).wait()

            @pl.when(g > 0)
            def _():
                pltpu.make_async_copy(
                    sbufs[b],
                    out_hbm.at[0, pl.ds(0, _S), pl.ds(0, _D)],
                    ssems[b],
                ).wait()

            @plsc.parallel_loop(0, _S, unroll=10)
            def _row(r):
                for c in range(_D // 16):
                    sbufs[b][r, pl.ds(c * 16, 16)] = (
                        gbufs[b][r, pl.ds(c * 16, 16)] * _SCALE
                    )

            pltpu.async_copy(
                sbufs[b],
                out_hbm.at[row0 + j, pl.ds(0, _S), pl.ds(0, _D)],
                ssems[b],
            )

            @pl.when(g < _NGRP - 1)
            def _():
                pltpu.async_copy(
                    table_hbm.at[idx_v.at[j + _NBUF]],
                    gbufs[b],
                    gsems[b],
                )

    for b in range(_NBUF):
        pltpu.make_async_copy(
            sbufs[b],
            out_hbm.at[0, pl.ds(0, _S), pl.ds(0, _D)],
            ssems[b],
        ).wait()


def kernel(inputs, embeddings):
    mesh = plsc.VectorSubcoreMesh(core_axis_name="c", subcore_axis_name="s")
    scratch = (
        [pltpu.VMEM((_ROWS_PER_W, _S), jnp.int32)]
        + [pltpu.VMEM((_S, _D), jnp.float32) for _ in range(_NBUF)]
        + [pltpu.VMEM((_S, _D), jnp.float32) for _ in range(_NBUF)]
        + [pltpu.SemaphoreType.DMA for _ in range(2 * _NBUF)]
    )
    f = pl.kernel(
        _sc_body,
        out_type=jax.ShapeDtypeStruct((_B, _SP, _DP), jnp.float32),
        mesh=mesh,
        scratch_types=scratch,
        compiler_params=pltpu.CompilerParams(use_tc_tiling_on_sc=False),
    )
    return f(inputs.astype(jnp.int32), embeddings)[:, :_S, :_D]
